# speedup vs baseline: 1.7934x; 1.7934x over previous
import functools

import jax
import jax.numpy as jnp
from jax.experimental import pallas as pl
from jax.experimental.pallas import tpu as pltpu


def _cvae_kernel(xcat_ref, img_ref, ut_ref, bt_ref, eps_ref,
                 emb_ref, off_ref,
                 t1_ref, b1_ref, t2_ref, b2_ref,
                 tw_ref, tb_ref,
                 e1w_ref, e1b_ref, e2w_ref, e2b_ref,
                 d1w_ref, d1b_ref, d2w_ref, d2b_ref,
                 out_ref, mu_ref, lv_ref,
                 *, latent_dim, n_tokens, feat_sizes):
    f32 = jnp.float32
    bf16 = jnp.bfloat16
    tb_rows = img_ref.shape[0]

    img = img_ref[...].astype(bf16)
    c1 = jnp.dot(img, t1_ref[...].astype(bf16), preferred_element_type=f32)
    c1 = jnp.maximum(c1 + b1_ref[...], 0.0)
    img_feat = jnp.dot(c1.astype(bf16), t2_ref[...].astype(bf16),
                       preferred_element_type=f32)
    img_feat = jnp.maximum(img_feat + b2_ref[...], 0.0)

    idx = xcat_ref[...] + off_ref[...]
    iota = jax.lax.broadcasted_iota(jnp.int32, (tb_rows, n_tokens), 1)
    emb = emb_ref[...]
    x0 = jnp.dot((iota == idx[:, 0:1]).astype(f32), emb,
                 preferred_element_type=f32)
    x1 = jnp.dot((iota == idx[:, 1:2]).astype(f32), emb,
                 preferred_element_type=f32)

    tw = tw_ref[...]
    tb = tb_ref[...]
    ut = jnp.dot(ut_ref[...], tw, preferred_element_type=f32) + tb
    bt = jnp.dot(bt_ref[...], tw, preferred_element_type=f32) + tb

    n_x, n_img, n_t = feat_sizes
    half = n_x // 2
    e1w = e1w_ref[...]
    h1 = jnp.dot(x0, e1w[0:half, :], preferred_element_type=f32)
    h1 = h1 + jnp.dot(x1, e1w[half:n_x, :], preferred_element_type=f32)
    h1 = h1 + jnp.dot(img_feat, e1w[n_x:n_x + n_img, :],
                      preferred_element_type=f32)
    h1 = h1 + jnp.dot(ut, e1w[n_x + n_img:n_x + n_img + n_t, :],
                      preferred_element_type=f32)
    h1 = h1 + jnp.dot(bt, e1w[n_x + n_img + n_t:, :],
                      preferred_element_type=f32)
    h1 = jnp.maximum(h1 + e1b_ref[...], 0.0)

    y = jnp.dot(h1, e2w_ref[...], preferred_element_type=f32) + e2b_ref[...]
    mu = y[:, :latent_dim]
    log_var = y[:, latent_dim:]
    mu_ref[...] = mu
    lv_ref[...] = log_var

    z = mu + eps_ref[...] * jnp.exp(0.5 * log_var)
    d1 = jnp.dot(z, d1w_ref[...], preferred_element_type=f32) + d1b_ref[...]
    d1 = jnp.maximum(d1, 0.0)
    out_ref[...] = jnp.sum(d1 * d2w_ref[...], axis=1, keepdims=True) + d2b_ref[...]


def kernel(x_cat, images, users_text, books_text, eps,
           emb_table, offsets, conv1_T, conv1_b, conv2_T, conv2_b,
           text_w, text_b, enc1_w, enc1_b, enc2_w, enc2_b,
           dec1_w, dec1_b, dec2_w, dec2_b):
    B = x_cat.shape[0]
    TB = 512
    while B % TB:
        TB //= 2
    grid = (B // TB,)

    latent_dim = eps.shape[1]
    n_tokens, embed_dim = emb_table.shape
    n_x = x_cat.shape[1] * embed_dim
    n_img = conv2_T.shape[1]
    n_t = text_w.shape[1]

    img_flat = images.reshape(B, -1)

    def act(n):
        return pl.BlockSpec((TB, n), lambda i: (i, 0))

    def const(a):
        return pl.BlockSpec(a.shape, lambda i: (0, 0))

    def row(v):
        return v.reshape(1, -1)

    off = offsets.reshape(1, -1)
    b1, b2 = row(conv1_b), row(conv2_b)
    tb_ = row(text_b)
    e1b, e2b = row(enc1_b), row(enc2_b)
    d1b, d2b = row(dec1_b), row(dec2_b)
    d2w = dec2_w.reshape(1, -1)

    body = functools.partial(_cvae_kernel,
                             latent_dim=latent_dim,
                             n_tokens=n_tokens,
                             feat_sizes=(n_x, n_img, n_t))

    out, mu, log_var = pl.pallas_call(
        body,
        out_shape=(jax.ShapeDtypeStruct((B, 1), jnp.float32),
                   jax.ShapeDtypeStruct((B, latent_dim), jnp.float32),
                   jax.ShapeDtypeStruct((B, latent_dim), jnp.float32)),
        grid=grid,
        in_specs=[act(x_cat.shape[1]), act(img_flat.shape[1]),
                  act(users_text.shape[1]), act(books_text.shape[1]),
                  act(latent_dim),
                  const(emb_table), const(off),
                  const(conv1_T), const(b1), const(conv2_T), const(b2),
                  const(text_w), const(tb_),
                  const(enc1_w), const(e1b), const(enc2_w), const(e2b),
                  const(dec1_w), const(d1b), const(d2w), const(d2b)],
        out_specs=(act(1), act(latent_dim), act(latent_dim)),
        compiler_params=pltpu.CompilerParams(
            dimension_semantics=("parallel",)),
    )(x_cat, img_flat, users_text, books_text, eps,
      emb_table, off, conv1_T, b1, conv2_T, b2, text_w, tb_,
      enc1_w, e1b, enc2_w, e2b, dec1_w, d1b, d2w, d2b)
    return out, mu, log_var
